# baseline (speedup 1.0000x reference)
import jax
import jax.numpy as jnp
from jax import lax
from jax.experimental import pallas as pl
from jax.experimental.pallas import tpu as pltpu, tpu_sc as plsc

_NC = 2
_NS = 16
_NW = _NC * _NS
_D = 128
_W = _D // 2
_L = 16
_C = 80


def _tree_sum(vs):
    while len(vs) > 1:
        vs = [a + b for a, b in zip(vs[::2], vs[1::2])]
    return vs[0]


def _edge_dot_kernel(n_edges):
    per_w = n_edges // _NW
    n_chunks = per_w // _C
    assert per_w % _C == 0 and n_chunks % 2 == 1

    mesh = plsc.VectorSubcoreMesh(core_axis_name="c", subcore_axis_name="s")

    @jax.jit
    def run(z, edge3):
        @pl.kernel(
            out_type=jax.ShapeDtypeStruct((n_edges,), jnp.float32),
            mesh=mesh,
            compiler_params=pltpu.CompilerParams(
                needs_layout_passes=False, use_tc_tiling_on_sc=False),
            scratch_types=[
                pltpu.VMEM((n_chunks, 2 * _C), jnp.int32),
                pltpu.VMEM((n_chunks, _C), jnp.int32),
                pltpu.VMEM((n_chunks, _C), jnp.int32),
                pltpu.VMEM((2, _C, _W), jnp.int32),
                pltpu.VMEM((2, _C, _W), jnp.int32),
                pltpu.VMEM((per_w,), jnp.float32),
                pltpu.VMEM((_L * _L,), jnp.float32),
                pltpu.SemaphoreType.DMA,
                pltpu.SemaphoreType.DMA,
                pltpu.SemaphoreType.DMA,
                pltpu.SemaphoreType.DMA,
            ],
        )
        def k(z_hbm, edge_hbm, out_hbm,
              eidx, sidx, didx, srows, drows, outv, tbuf, ss0, ss1, sd0, sd1):
            wid = lax.axis_index("s") * _NC + lax.axis_index("c")
            pltpu.sync_copy(edge_hbm.at[wid], eidx)

            lane2 = lax.iota(jnp.int32, _L) * 2
            def deint(ci, _):
                row = eidx.at[ci]
                for v in range(_C // _L):
                    sidx[ci, pl.ds(v * _L, _L)] = plsc.load_gather(
                        row, [lane2 + (2 * v * _L)])
                    didx[ci, pl.ds(v * _L, _L)] = plsc.load_gather(
                        row, [lane2 + (2 * v * _L + 1)])
                return 0
            lax.fori_loop(0, n_chunks, deint, 0)

            ssems = (ss0, ss1)
            dsems = (sd0, sd1)

            def start(i, b):
                pltpu.async_copy(z_hbm.at[sidx.at[i]], srows.at[b], ssems[b])
                pltpu.async_copy(z_hbm.at[didx.at[i]], drows.at[b], dsems[b])

            def wait(b):
                dummy = z_hbm.at[pl.ds(0, _C)]
                pltpu.make_async_copy(dummy, srows.at[b], ssems[b]).wait()
                pltpu.make_async_copy(dummy, drows.at[b], dsems[b]).wait()

            def compute(g, b):
                sr = srows.at[b]
                dr = drows.at[b]

                def group(gi, _):
                    eb = gi * _L
                    for j in range(_L):
                        e = eb + j
                        ps = []
                        for t in range(_D // (2 * _L)):
                            a = plsc.bitcast(sr[e, pl.ds(t * _L, _L)],
                                             jnp.bfloat16)
                            b_ = plsc.bitcast(dr[e, pl.ds(t * _L, _L)],
                                              jnp.bfloat16)
                            p0, p1 = plsc.unpack(
                                a * b_, format=plsc.PackFormat.INTERLEAVED,
                                preferred_element_type=jnp.float32)
                            ps += [p0, p1]
                        tbuf[pl.ds(j * _L, _L)] = _tree_sum(ps)
                    colidx = lax.iota(jnp.int32, _L) * _L
                    cols = [plsc.load_gather(tbuf, [colidx + l])
                            for l in range(_L)]
                    outv[pl.ds(g * _C + eb, _L)] = _tree_sum(cols)
                    return 0

                lax.fori_loop(0, _C // _L, group, 0)

            start(0, 0)

            def outer(t, _):
                g0 = t * 2
                for b in range(2):
                    g = g0 + b
                    wait(b)

                    @pl.when(g + 1 < n_chunks)
                    def _():
                        start(g + 1, 1 - b)

                    compute(g, b)
                return 0

            lax.fori_loop(0, (n_chunks - 1) // 2, outer, 0)
            wait(0)
            compute(n_chunks - 1, 0)
            pltpu.sync_copy(outv, out_hbm.at[pl.ds(wid * per_w, per_w)])

        return k(z, edge3)

    return run


def kernel(z, edge):
    n_edges = edge.shape[0]
    per_w = n_edges // _NW
    edge3 = edge.astype(jnp.int32).reshape(_NW, per_w // _C, 2 * _C)
    zi = lax.bitcast_convert_type(
        z.astype(jnp.bfloat16).reshape(z.shape[0], z.shape[1] // 2, 2),
        jnp.int32)
    return _edge_dot_kernel(n_edges)(zi, edge3)


# speedup vs baseline: 1.8088x; 1.8088x over previous
import jax
import jax.numpy as jnp
from jax import lax
from jax.experimental import pallas as pl
from jax.experimental.pallas import tpu as pltpu, tpu_sc as plsc

_NC = 2
_NS = 16
_NW = _NC * _NS
_D = 128
_W = _D // 2
_L = 16
_C = 80


def _tree_sum(vs):
    while len(vs) > 1:
        vs = [a + b for a, b in zip(vs[::2], vs[1::2])]
    return vs[0]


def _edge_dot_kernel(n_edges):
    per_w = n_edges // _NW
    n_chunks = per_w // _C
    assert per_w % _C == 0 and n_chunks % 2 == 1

    mesh = plsc.VectorSubcoreMesh(core_axis_name="c", subcore_axis_name="s")

    @jax.jit
    def run(z, src2, dst2):
        @pl.kernel(
            out_type=jax.ShapeDtypeStruct((n_edges,), jnp.float32),
            mesh=mesh,
            compiler_params=pltpu.CompilerParams(
                needs_layout_passes=False, use_tc_tiling_on_sc=False),
            scratch_types=[
                pltpu.VMEM((n_chunks, _C), jnp.int32),
                pltpu.VMEM((n_chunks, _C), jnp.int32),
                pltpu.VMEM((2, _C, _W), jnp.int32),
                pltpu.VMEM((2, _C, _W), jnp.int32),
                pltpu.VMEM((per_w,), jnp.float32),
                pltpu.VMEM((_L * _L,), jnp.float32),
                pltpu.SemaphoreType.DMA,
                pltpu.SemaphoreType.DMA,
                pltpu.SemaphoreType.DMA,
                pltpu.SemaphoreType.DMA,
            ],
        )
        def k(z_hbm, src_hbm, dst_hbm, out_hbm,
              sidx, didx, srows, drows, outv, tbuf, ss0, ss1, sd0, sd1):
            wid = lax.axis_index("s") * _NC + lax.axis_index("c")
            pltpu.sync_copy(src_hbm.at[wid], sidx)
            pltpu.sync_copy(dst_hbm.at[wid], didx)

            ssems = (ss0, ss1)
            dsems = (sd0, sd1)

            def start(i, b):
                pltpu.async_copy(z_hbm.at[sidx.at[i]], srows.at[b], ssems[b])
                pltpu.async_copy(z_hbm.at[didx.at[i]], drows.at[b], dsems[b])

            def wait(b):
                dummy = z_hbm.at[pl.ds(0, _C)]
                pltpu.make_async_copy(dummy, srows.at[b], ssems[b]).wait()
                pltpu.make_async_copy(dummy, drows.at[b], dsems[b]).wait()

            def compute(g, b):
                sr = srows.at[b]
                dr = drows.at[b]

                def group(gi, _):
                    eb = gi * _L

                    def edge(j, _):
                        e = eb + j
                        ps = []
                        for t in range(_D // (2 * _L)):
                            a = plsc.bitcast(sr[e, pl.ds(t * _L, _L)],
                                             jnp.bfloat16)
                            b_ = plsc.bitcast(dr[e, pl.ds(t * _L, _L)],
                                              jnp.bfloat16)
                            p0, p1 = plsc.unpack(
                                a * b_, format=plsc.PackFormat.INTERLEAVED,
                                preferred_element_type=jnp.float32)
                            ps += [p0, p1]
                        tbuf[pl.ds(j * _L, _L)] = _tree_sum(ps)
                        return 0

                    lax.fori_loop(0, _L, edge, 0, unroll=2)
                    colidx = lax.iota(jnp.int32, _L) * _L
                    cols = [plsc.load_gather(tbuf, [colidx + l])
                            for l in range(_L)]
                    outv[pl.ds(g * _C + eb, _L)] = _tree_sum(cols)
                    return 0

                lax.fori_loop(0, _C // _L, group, 0)

            start(0, 0)

            def outer(t, _):
                g0 = t * 2
                for b in range(2):
                    g = g0 + b
                    wait(b)

                    @pl.when(g + 1 < n_chunks)
                    def _():
                        start(g + 1, 1 - b)

                    compute(g, b)
                return 0

            lax.fori_loop(0, (n_chunks - 1) // 2, outer, 0)
            wait(0)
            compute(n_chunks - 1, 0)
            pltpu.sync_copy(outv, out_hbm.at[pl.ds(wid * per_w, per_w)])

        return k(z, src2, dst2)

    return run


def kernel(z, edge):
    n_edges = edge.shape[0]
    per_w = n_edges // _NW
    src2 = edge[:, 0].astype(jnp.int32).reshape(_NW, per_w // _C, _C)
    dst2 = edge[:, 1].astype(jnp.int32).reshape(_NW, per_w // _C, _C)
    zi = lax.bitcast_convert_type(
        z.astype(jnp.bfloat16).reshape(z.shape[0], z.shape[1] // 2, 2),
        jnp.int32)
    return _edge_dot_kernel(n_edges)(zi, src2, dst2)
